# Initial kernel scaffold; baseline (speedup 1.0000x reference)
import jax
import jax.numpy as jnp
from jax.experimental import pallas as pl


def kernel(x, edge_index, bn1_g, bn1_b, W_l1, b_l1, W_r1, b_r1, bn2_g, bn2_b, W_l2, b_l2, W_r2, b_r2, W_out, b_out):
    raise NotImplementedError("write your pallas kernel here")


# speedup vs baseline: 5.1208x; 5.1208x over previous
import jax
import jax.numpy as jnp
from jax import lax
from jax.experimental import pallas as pl
from jax.experimental.pallas import tpu as pltpu
from jax.experimental.pallas import tpu_sc as plsc

N = 10000
D = 128
E = 320000
HD = D // 2

NC = 2
NS = 16
EPT = E // NS
CHUNK = 80
NCHUNK = EPT // CHUNK
CW = 16
NP = 10240
RPT = NP // NS


def _make_sc_agg(with_counts: bool):
    mesh = plsc.VectorSubcoreMesh(core_axis_name="c", subcore_axis_name="s")
    out_type = [jax.ShapeDtypeStruct((NC, NP, HD), jnp.float32)]
    scratch = [
        pltpu.VMEM((NCHUNK, CHUNK), jnp.int32),
        pltpu.VMEM((NCHUNK, CHUNK), jnp.int32),
        pltpu.VMEM((CHUNK, HD), jnp.float32),
        pltpu.VMEM_SHARED((NP, HD), jnp.float32),
        pltpu.SemaphoreType.DMA,
    ]
    if with_counts:
        out_type.append(jax.ShapeDtypeStruct((NP, CW), jnp.float32))
        scratch.insert(3, pltpu.VMEM((CHUNK, CW), jnp.float32))
        scratch.insert(5, pltpu.VMEM_SHARED((NP, CW), jnp.float32))

    def body(*refs):
        if with_counts:
            (h_hbm, srcx_hbm, dstx_hbm, z_hbm, zc_hbm, ones_hbm,
             agg_hbm, cnt_hbm,
             src_v, dst_v, rows_v, ones_v, acc_sh, cnt_sh, sem) = refs
        else:
            (h_hbm, srcx_hbm, dstx_hbm, z_hbm,
             agg_hbm,
             src_v, dst_v, rows_v, acc_sh, sem) = refs

        cid = lax.axis_index("c")
        sid = lax.axis_index("s")
        base = sid * RPT

        pltpu.sync_copy(z_hbm.at[pl.ds(base, RPT)], acc_sh.at[pl.ds(base, RPT)])
        if with_counts:
            @pl.when(cid == 0)
            def _():
                pltpu.sync_copy(zc_hbm.at[pl.ds(base, RPT)],
                                cnt_sh.at[pl.ds(base, RPT)])
                pltpu.sync_copy(ones_hbm, ones_v)

        pltpu.sync_copy(srcx_hbm.at[cid, sid], src_v)
        pltpu.sync_copy(dstx_hbm.at[sid], dst_v)
        plsc.subcore_barrier()

        def step(i, carry):
            pltpu.async_copy(h_hbm.at[src_v.at[i]], rows_v, sem).wait()
            pltpu.sync_copy(rows_v, acc_sh.at[dst_v.at[i]], add=True)
            if with_counts:
                @pl.when(cid == 0)
                def _():
                    pltpu.sync_copy(ones_v, cnt_sh.at[dst_v.at[i]], add=True)
            return carry

        lax.fori_loop(0, NCHUNK, step, 0)

        plsc.subcore_barrier()
        pltpu.sync_copy(acc_sh.at[pl.ds(base, RPT)],
                        agg_hbm.at[cid, pl.ds(base, RPT)])
        if with_counts:
            @pl.when(cid == 0)
            def _():
                pltpu.sync_copy(cnt_sh.at[pl.ds(base, RPT)],
                                cnt_hbm.at[pl.ds(base, RPT)])

    return pl.kernel(body, out_type=tuple(out_type), mesh=mesh,
                     scratch_types=scratch,
                     compiler_params=pltpu.CompilerParams(
                         use_tc_tiling_on_sc=False))


_sc_agg_counts = _make_sc_agg(True)
_sc_agg = _make_sc_agg(False)


def _bn_body(x_ref, g_ref, b_ref, o_ref):
    x = x_ref[...]
    m = jnp.mean(x, axis=0, keepdims=True)
    xc = x - m
    v = jnp.mean(xc * xc, axis=0, keepdims=True)
    o_ref[...] = xc * lax.rsqrt(v + 1e-5) * g_ref[...] + b_ref[...]


_bn = pl.pallas_call(
    _bn_body, out_shape=jax.ShapeDtypeStruct((N, D), jnp.float32))


def _dense1_body(agg_ref, cnt_ref, h_ref, wlT_ref, bl_ref, wrT_ref, br_ref,
                 g_ref, b_ref, o_ref):
    agg = jnp.concatenate([agg_ref[0, :N], agg_ref[1, :N]], axis=1)
    cnt = cnt_ref[:N, 0:1]
    mean = agg / jnp.maximum(cnt, 1.0)
    t = (jnp.dot(mean, wlT_ref[...], preferred_element_type=jnp.float32)
         + bl_ref[...]
         + jnp.dot(h_ref[...], wrT_ref[...], preferred_element_type=jnp.float32)
         + br_ref[...])
    t = jnp.maximum(t, 0.0)
    m = jnp.mean(t, axis=0, keepdims=True)
    tc = t - m
    v = jnp.mean(tc * tc, axis=0, keepdims=True)
    o_ref[...] = tc * lax.rsqrt(v + 1e-5) * g_ref[...] + b_ref[...]


_dense1 = pl.pallas_call(
    _dense1_body, out_shape=jax.ShapeDtypeStruct((N, D), jnp.float32))


def _dense2_body(agg_ref, cnt_ref, h_ref, wlT_ref, bl_ref, wrT_ref, br_ref,
                 woT_ref, bo_ref, o_ref):
    agg = jnp.concatenate([agg_ref[0, :N], agg_ref[1, :N]], axis=1)
    cnt = cnt_ref[:N, 0:1]
    mean = agg / jnp.maximum(cnt, 1.0)
    t = (jnp.dot(mean, wlT_ref[...], preferred_element_type=jnp.float32)
         + bl_ref[...]
         + jnp.dot(h_ref[...], wrT_ref[...], preferred_element_type=jnp.float32)
         + br_ref[...])
    t = jnp.maximum(t, 0.0)
    z = jnp.dot(t, woT_ref[...], preferred_element_type=jnp.float32) + bo_ref[...]
    o_ref[...] = jax.nn.sigmoid(z)


_dense2 = pl.pallas_call(
    _dense2_body, out_shape=jax.ShapeDtypeStruct((N, 1), jnp.float32))


def kernel(x, edge_index, bn1_g, bn1_b, W_l1, b_l1, W_r1, b_r1,
           bn2_g, bn2_b, W_l2, b_l2, W_r2, b_r2, W_out, b_out):
    src2 = (2 * edge_index[0].astype(jnp.int32)).reshape(NS, NCHUNK, CHUNK)
    srcx = jnp.stack([src2, src2 + 1])
    dstx = edge_index[1].astype(jnp.int32).reshape(NS, NCHUNK, CHUNK)
    z_nd = jnp.zeros((NP, HD), jnp.float32)
    z_nc = jnp.zeros((NP, CW), jnp.float32)
    ones_cw = jnp.ones((CHUNK, CW), jnp.float32)

    h = _bn(x, bn1_g.reshape(1, D), bn1_b.reshape(1, D))
    aggp, cnt = _sc_agg_counts(h.reshape(2 * N, HD), srcx, dstx,
                               z_nd, z_nc, ones_cw)
    h2 = _dense1(aggp, cnt, h, W_l1.T, b_l1.reshape(1, D),
                 W_r1.T, b_r1.reshape(1, D),
                 bn2_g.reshape(1, D), bn2_b.reshape(1, D))
    (aggp2,) = _sc_agg(h2.reshape(2 * N, HD), srcx, dstx, z_nd)
    out = _dense2(aggp2, cnt, h2, W_l2.T, b_l2.reshape(1, D),
                  W_r2.T, b_r2.reshape(1, D),
                  W_out.T, b_out.reshape(1, 1))
    return out.reshape(N)


# speedup vs baseline: 8.5141x; 1.6626x over previous
import jax
import jax.numpy as jnp
from jax import lax
from jax.experimental import pallas as pl
from jax.experimental.pallas import tpu as pltpu
from jax.experimental.pallas import tpu_sc as plsc

N = 10000
D = 128
E = 320000
HD = D // 2

NC = 2
NS = 16
EPT = E // NS
CHUNK = 80
NCHUNK = EPT // CHUNK
CW = 16
NP = 10240
RPT = NP // NS


def _make_sc_agg(with_counts: bool):
    nbuf = 2 if with_counts else 5
    ngrp = NCHUNK // nbuf
    mesh = plsc.VectorSubcoreMesh(core_axis_name="c", subcore_axis_name="s")
    out_type = [jax.ShapeDtypeStruct((NC, NP, HD), jnp.float32)]
    scratch = [
        pltpu.VMEM((NCHUNK, CHUNK), jnp.int32),
        pltpu.VMEM((NCHUNK, CHUNK), jnp.int32),
        pltpu.VMEM((nbuf, CHUNK, HD), jnp.float32),
        pltpu.VMEM_SHARED((NP, HD), jnp.float32),
        pltpu.SemaphoreType.DMA,
        pltpu.SemaphoreType.DMA,
    ]
    if with_counts:
        out_type.append(jax.ShapeDtypeStruct((NC, NP, CW), jnp.float32))
        scratch.insert(3, pltpu.VMEM((CHUNK, CW), jnp.float32))
        scratch.insert(5, pltpu.VMEM_SHARED((NP, CW), jnp.float32))
        scratch.append(pltpu.SemaphoreType.DMA)

    def body(*refs):
        if with_counts:
            (h_hbm, srcx_hbm, dstx_hbm, z_hbm, zc_hbm, ones_hbm,
             agg_hbm, cnt_hbm,
             src_v, dst_v, rows_v, ones_v, acc_sh, cnt_sh,
             sem_g, sem_s, sem_c) = refs
        else:
            (h_hbm, srcx_hbm, dstx_hbm, z_hbm,
             agg_hbm,
             src_v, dst_v, rows_v, acc_sh, sem_g, sem_s) = refs

        cid = lax.axis_index("c")
        sid = lax.axis_index("s")
        base = sid * RPT

        pltpu.sync_copy(z_hbm.at[pl.ds(base, RPT)], acc_sh.at[pl.ds(base, RPT)])
        if with_counts:
            pltpu.sync_copy(zc_hbm.at[pl.ds(base, RPT)],
                            cnt_sh.at[pl.ds(base, RPT)])
            pltpu.sync_copy(ones_hbm, ones_v)

        pltpu.sync_copy(srcx_hbm.at[cid, sid], src_v)
        pltpu.sync_copy(dstx_hbm.at[sid], dst_v)
        plsc.subcore_barrier()

        def step(j, carry):
            c0 = j * nbuf
            gathers = []
            for b in range(nbuf):
                gathers.append(pltpu.async_copy(
                    h_hbm.at[src_v.at[c0 + b]], rows_v.at[b], sem_g))
            scatters = []
            for b in range(nbuf):
                gathers[b].wait()
                scatters.append(pltpu.async_copy(
                    rows_v.at[b], acc_sh.at[dst_v.at[c0 + b]], sem_s,
                    add=True))
            if with_counts:
                @pl.when((cid == 0) == (j < ngrp // 2))
                def _():
                    cs = [pltpu.async_copy(
                        ones_v, cnt_sh.at[dst_v.at[c0 + b]], sem_c,
                        add=True) for b in range(nbuf)]
                    for c in cs:
                        c.wait()
            for s in scatters:
                s.wait()
            return carry

        lax.fori_loop(0, ngrp, step, 0)

        plsc.subcore_barrier()
        pltpu.sync_copy(acc_sh.at[pl.ds(base, RPT)],
                        agg_hbm.at[cid, pl.ds(base, RPT)])
        if with_counts:
            pltpu.sync_copy(cnt_sh.at[pl.ds(base, RPT)],
                            cnt_hbm.at[cid, pl.ds(base, RPT)])

    return pl.kernel(body, out_type=tuple(out_type), mesh=mesh,
                     scratch_types=scratch,
                     compiler_params=pltpu.CompilerParams(
                         use_tc_tiling_on_sc=False))


_sc_agg_counts = _make_sc_agg(True)
_sc_agg = _make_sc_agg(False)


def _bn_body(x_ref, g_ref, b_ref, o_ref):
    x = x_ref[...]
    m = jnp.mean(x, axis=0, keepdims=True)
    xc = x - m
    v = jnp.mean(xc * xc, axis=0, keepdims=True)
    o_ref[...] = xc * lax.rsqrt(v + 1e-5) * g_ref[...] + b_ref[...]


_bn = pl.pallas_call(
    _bn_body, out_shape=jax.ShapeDtypeStruct((N, D), jnp.float32))


def _dense1_body(agg_ref, cnt_ref, h_ref, wlT_ref, bl_ref, wrT_ref, br_ref,
                 g_ref, b_ref, o_ref):
    agg = jnp.concatenate([agg_ref[0, :N], agg_ref[1, :N]], axis=1)
    cnt = cnt_ref[0, :N, 0:1] + cnt_ref[1, :N, 0:1]
    mean = agg / jnp.maximum(cnt, 1.0)
    t = (jnp.dot(mean, wlT_ref[...], preferred_element_type=jnp.float32)
         + bl_ref[...]
         + jnp.dot(h_ref[...], wrT_ref[...], preferred_element_type=jnp.float32)
         + br_ref[...])
    t = jnp.maximum(t, 0.0)
    m = jnp.mean(t, axis=0, keepdims=True)
    tc = t - m
    v = jnp.mean(tc * tc, axis=0, keepdims=True)
    o_ref[...] = tc * lax.rsqrt(v + 1e-5) * g_ref[...] + b_ref[...]


_dense1 = pl.pallas_call(
    _dense1_body, out_shape=jax.ShapeDtypeStruct((N, D), jnp.float32))


def _dense2_body(agg_ref, cnt_ref, h_ref, wlT_ref, bl_ref, wrT_ref, br_ref,
                 woT_ref, bo_ref, o_ref):
    agg = jnp.concatenate([agg_ref[0, :N], agg_ref[1, :N]], axis=1)
    cnt = cnt_ref[0, :N, 0:1] + cnt_ref[1, :N, 0:1]
    mean = agg / jnp.maximum(cnt, 1.0)
    t = (jnp.dot(mean, wlT_ref[...], preferred_element_type=jnp.float32)
         + bl_ref[...]
         + jnp.dot(h_ref[...], wrT_ref[...], preferred_element_type=jnp.float32)
         + br_ref[...])
    t = jnp.maximum(t, 0.0)
    z = jnp.dot(t, woT_ref[...], preferred_element_type=jnp.float32) + bo_ref[...]
    o_ref[...] = jax.nn.sigmoid(z)


_dense2 = pl.pallas_call(
    _dense2_body, out_shape=jax.ShapeDtypeStruct((N, 1), jnp.float32))


def kernel(x, edge_index, bn1_g, bn1_b, W_l1, b_l1, W_r1, b_r1,
           bn2_g, bn2_b, W_l2, b_l2, W_r2, b_r2, W_out, b_out):
    src2 = (2 * edge_index[0].astype(jnp.int32)).reshape(NS, NCHUNK, CHUNK)
    srcx = jnp.stack([src2, src2 + 1])
    dstx = edge_index[1].astype(jnp.int32).reshape(NS, NCHUNK, CHUNK)
    z_nd = jnp.zeros((NP, HD), jnp.float32)
    z_nc = jnp.zeros((NP, CW), jnp.float32)
    ones_cw = jnp.ones((CHUNK, CW), jnp.float32)

    h = _bn(x, bn1_g.reshape(1, D), bn1_b.reshape(1, D))
    aggp, cnt = _sc_agg_counts(h.reshape(2 * N, HD), srcx, dstx,
                               z_nd, z_nc, ones_cw)
    h2 = _dense1(aggp, cnt, h, W_l1.T, b_l1.reshape(1, D),
                 W_r1.T, b_r1.reshape(1, D),
                 bn2_g.reshape(1, D), bn2_b.reshape(1, D))
    (aggp2,) = _sc_agg(h2.reshape(2 * N, HD), srcx, dstx, z_nd)
    out = _dense2(aggp2, cnt, h2, W_l2.T, b_l2.reshape(1, D),
                  W_r2.T, b_r2.reshape(1, D),
                  W_out.T, b_out.reshape(1, 1))
    return out.reshape(N)


# speedup vs baseline: 9.1603x; 1.0759x over previous
import jax
import jax.numpy as jnp
from jax import lax
from jax.experimental import pallas as pl
from jax.experimental.pallas import tpu as pltpu
from jax.experimental.pallas import tpu_sc as plsc

N = 10000
D = 128
E = 320000
HD = D // 2

NC = 2
NS = 16
EPT = E // NS
CHUNK = 40
NCHUNK = EPT // CHUNK
CW = 16
NP = 10240
RPT = NP // NS


def _make_sc_agg(with_counts: bool):
    nbuf = 5 if with_counts else 10
    ngrp = NCHUNK // nbuf
    mesh = plsc.VectorSubcoreMesh(core_axis_name="c", subcore_axis_name="s")
    out_type = [jax.ShapeDtypeStruct((NC, NP, HD), jnp.float32)]
    scratch = [
        pltpu.VMEM((EPT,), jnp.int32),
        pltpu.VMEM((NCHUNK, CHUNK), jnp.int32),
        pltpu.VMEM((nbuf, CHUNK, HD), jnp.float32),
        pltpu.VMEM_SHARED((NP, HD), jnp.float32),
        pltpu.SemaphoreType.DMA,
        pltpu.SemaphoreType.DMA,
    ]
    if with_counts:
        out_type.append(jax.ShapeDtypeStruct((NC, NP, CW), jnp.float32))
        scratch.insert(3, pltpu.VMEM((CHUNK, CW), jnp.float32))
        scratch.insert(5, pltpu.VMEM_SHARED((NP, CW), jnp.float32))
        scratch.append(pltpu.SemaphoreType.DMA)

    def body(*refs):
        if with_counts:
            (h_hbm, srcx_hbm, dstx_hbm, z_hbm, zc_hbm, ones_hbm,
             agg_hbm, cnt_hbm,
             src_v, dst_v, rows_v, ones_v, acc_sh, cnt_sh,
             sem_g, sem_s, sem_c) = refs
        else:
            (h_hbm, srcx_hbm, dstx_hbm, z_hbm,
             agg_hbm,
             src_v, dst_v, rows_v, acc_sh, sem_g, sem_s) = refs

        cid = lax.axis_index("c")
        sid = lax.axis_index("s")
        base = sid * RPT

        pltpu.sync_copy(z_hbm.at[pl.ds(base, RPT)], acc_sh.at[pl.ds(base, RPT)])
        if with_counts:
            pltpu.sync_copy(zc_hbm.at[pl.ds(base, RPT)],
                            cnt_sh.at[pl.ds(base, RPT)])
            pltpu.sync_copy(ones_hbm, ones_v)

        pltpu.sync_copy(srcx_hbm.at[sid], src_v)
        pltpu.sync_copy(dstx_hbm.at[sid], dst_v)

        def xform(i, carry):
            s = src_v[pl.ds(i * 16, 16)]
            src_v[pl.ds(i * 16, 16)] = s + s + cid
            return carry

        lax.fori_loop(0, EPT // 16, xform, 0)
        plsc.subcore_barrier()

        def step(j, carry):
            c0 = j * nbuf
            gathers = []
            for b in range(nbuf):
                gathers.append(pltpu.async_copy(
                    h_hbm.at[src_v.at[pl.ds((c0 + b) * CHUNK, CHUNK)]],
                    rows_v.at[b], sem_g))
            scatters = []
            for b in range(nbuf):
                gathers[b].wait()
                scatters.append(pltpu.async_copy(
                    rows_v.at[b], acc_sh.at[dst_v.at[c0 + b]], sem_s,
                    add=True))
            if with_counts:
                @pl.when((cid == 0) == (j < ngrp // 2))
                def _():
                    cs = [pltpu.async_copy(
                        ones_v, cnt_sh.at[dst_v.at[c0 + b]], sem_c,
                        add=True) for b in range(nbuf)]
                    for c in cs:
                        c.wait()
            for s in scatters:
                s.wait()
            return carry

        lax.fori_loop(0, ngrp, step, 0)

        plsc.subcore_barrier()
        pltpu.sync_copy(acc_sh.at[pl.ds(base, RPT)],
                        agg_hbm.at[cid, pl.ds(base, RPT)])
        if with_counts:
            pltpu.sync_copy(cnt_sh.at[pl.ds(base, RPT)],
                            cnt_hbm.at[cid, pl.ds(base, RPT)])

    return pl.kernel(body, out_type=tuple(out_type), mesh=mesh,
                     scratch_types=scratch,
                     compiler_params=pltpu.CompilerParams(
                         use_tc_tiling_on_sc=False))


_sc_agg_counts = _make_sc_agg(True)
_sc_agg = _make_sc_agg(False)


def _bn_body(x_ref, g_ref, b_ref, o_ref):
    x = x_ref[...]
    m = jnp.mean(x, axis=0, keepdims=True)
    xc = x - m
    v = jnp.mean(xc * xc, axis=0, keepdims=True)
    o_ref[...] = xc * lax.rsqrt(v + 1e-5) * g_ref[...] + b_ref[...]


_bn = pl.pallas_call(
    _bn_body, out_shape=jax.ShapeDtypeStruct((N, D), jnp.float32))


def _dense1_body(agg_ref, cnt_ref, h_ref, wlT_ref, bl_ref, wrT_ref, br_ref,
                 g_ref, b_ref, o_ref):
    agg = jnp.concatenate([agg_ref[0, :N], agg_ref[1, :N]], axis=1)
    cnt = cnt_ref[0, :N, 0:1] + cnt_ref[1, :N, 0:1]
    mean = agg / jnp.maximum(cnt, 1.0)
    t = (jnp.dot(mean, wlT_ref[...], preferred_element_type=jnp.float32)
         + bl_ref[...]
         + jnp.dot(h_ref[...], wrT_ref[...], preferred_element_type=jnp.float32)
         + br_ref[...])
    t = jnp.maximum(t, 0.0)
    m = jnp.mean(t, axis=0, keepdims=True)
    tc = t - m
    v = jnp.mean(tc * tc, axis=0, keepdims=True)
    o_ref[...] = tc * lax.rsqrt(v + 1e-5) * g_ref[...] + b_ref[...]


_dense1 = pl.pallas_call(
    _dense1_body, out_shape=jax.ShapeDtypeStruct((N, D), jnp.float32))


def _dense2_body(agg_ref, cnt_ref, h_ref, wlT_ref, bl_ref, wrT_ref, br_ref,
                 woT_ref, bo_ref, o_ref):
    agg = jnp.concatenate([agg_ref[0, :N], agg_ref[1, :N]], axis=1)
    cnt = cnt_ref[0, :N, 0:1] + cnt_ref[1, :N, 0:1]
    mean = agg / jnp.maximum(cnt, 1.0)
    t = (jnp.dot(mean, wlT_ref[...], preferred_element_type=jnp.float32)
         + bl_ref[...]
         + jnp.dot(h_ref[...], wrT_ref[...], preferred_element_type=jnp.float32)
         + br_ref[...])
    t = jnp.maximum(t, 0.0)
    z = jnp.dot(t, woT_ref[...], preferred_element_type=jnp.float32) + bo_ref[...]
    o_ref[...] = jax.nn.sigmoid(z)


_dense2 = pl.pallas_call(
    _dense2_body, out_shape=jax.ShapeDtypeStruct((N, 1), jnp.float32))


def kernel(x, edge_index, bn1_g, bn1_b, W_l1, b_l1, W_r1, b_r1,
           bn2_g, bn2_b, W_l2, b_l2, W_r2, b_r2, W_out, b_out):
    srcx = edge_index[0].astype(jnp.int32).reshape(NS, EPT)
    dstx = edge_index[1].astype(jnp.int32).reshape(NS, NCHUNK, CHUNK)
    z_nd = jnp.zeros((NP, HD), jnp.float32)
    z_nc = jnp.zeros((NP, CW), jnp.float32)
    ones_cw = jnp.ones((CHUNK, CW), jnp.float32)

    h = _bn(x, bn1_g.reshape(1, D), bn1_b.reshape(1, D))
    aggp, cnt = _sc_agg_counts(h.reshape(2 * N, HD), srcx, dstx,
                               z_nd, z_nc, ones_cw)
    h2 = _dense1(aggp, cnt, h, W_l1.T, b_l1.reshape(1, D),
                 W_r1.T, b_r1.reshape(1, D),
                 bn2_g.reshape(1, D), bn2_b.reshape(1, D))
    (aggp2,) = _sc_agg(h2.reshape(2 * N, HD), srcx, dstx, z_nd)
    out = _dense2(aggp2, cnt, h2, W_l2.T, b_l2.reshape(1, D),
                  W_r2.T, b_r2.reshape(1, D),
                  W_out.T, b_out.reshape(1, 1))
    return out.reshape(N)


# speedup vs baseline: 10.0887x; 1.1013x over previous
import jax
import jax.numpy as jnp
from jax import lax
from jax.experimental import pallas as pl
from jax.experimental.pallas import tpu as pltpu
from jax.experimental.pallas import tpu_sc as plsc

N = 10000
D = 128
E = 320000
HD = D // 2

NC = 2
NS = 16
EPT = E // NS
CHUNK = 40
NCHUNK = EPT // CHUNK
CW = 16
NP = 10240
RPT = NP // NS


def _make_sc_agg(with_counts: bool):
    nbuf = 10
    ngrp = NCHUNK // nbuf
    mesh = plsc.VectorSubcoreMesh(core_axis_name="c", subcore_axis_name="s")
    out_type = [jax.ShapeDtypeStruct((NC, NP, HD), jnp.float32)]
    scratch = [
        pltpu.VMEM((EPT,), jnp.int32),
        pltpu.VMEM((EPT,), jnp.int32),
        pltpu.VMEM((nbuf, CHUNK, HD), jnp.float32),
        pltpu.VMEM_SHARED((NP, HD), jnp.float32),
        pltpu.SemaphoreType.DMA,
        pltpu.SemaphoreType.DMA,
    ]
    if with_counts:
        out_type.append(jax.ShapeDtypeStruct((NS, NP), jnp.float32))
        scratch.insert(3, pltpu.VMEM((NP,), jnp.float32))

    def body(*refs):
        if with_counts:
            (h_hbm, srcx_hbm, dstx_hbm, z_hbm,
             agg_hbm, cnt_hbm,
             src_v, dst_v, rows_v, cnt_t, acc_sh,
             sem_g, sem_s) = refs
        else:
            (h_hbm, srcx_hbm, dstx_hbm, z_hbm,
             agg_hbm,
             src_v, dst_v, rows_v, acc_sh, sem_g, sem_s) = refs

        cid = lax.axis_index("c")
        sid = lax.axis_index("s")
        base = sid * RPT

        pltpu.sync_copy(z_hbm.at[pl.ds(base, RPT)], acc_sh.at[pl.ds(base, RPT)])

        pltpu.sync_copy(srcx_hbm.at[sid], src_v)
        pltpu.sync_copy(dstx_hbm.at[sid], dst_v)

        def xform(i, carry):
            s = src_v[pl.ds(i * 16, 16)]
            src_v[pl.ds(i * 16, 16)] = s + s + cid
            return carry

        lax.fori_loop(0, EPT // 16, xform, 0)

        if with_counts:
            @pl.when(cid == 0)
            def _():
                def czero(i, carry):
                    cnt_t[pl.ds(i * 16, 16)] = jnp.zeros((16,), jnp.float32)
                    return carry

                lax.fori_loop(0, NP // 16, czero, 0)

                def count(i, carry):
                    idx = dst_v[pl.ds(i * 16, 16)]
                    plsc.addupdate_scatter(cnt_t, [idx],
                                           jnp.ones((16,), jnp.float32))
                    return carry

                lax.fori_loop(0, EPT // 16, count, 0)
                pltpu.sync_copy(cnt_t, cnt_hbm.at[sid])

        plsc.subcore_barrier()

        def step(j, carry):
            c0 = j * nbuf
            gathers = []
            for b in range(nbuf):
                gathers.append(pltpu.async_copy(
                    h_hbm.at[src_v.at[pl.ds((c0 + b) * CHUNK, CHUNK)]],
                    rows_v.at[b], sem_g))
            scatters = []
            for b in range(nbuf):
                gathers[b].wait()
                scatters.append(pltpu.async_copy(
                    rows_v.at[b],
                    acc_sh.at[dst_v.at[pl.ds((c0 + b) * CHUNK, CHUNK)]],
                    sem_s, add=True))
            for s in scatters:
                s.wait()
            return carry

        lax.fori_loop(0, ngrp, step, 0)

        plsc.subcore_barrier()
        pltpu.sync_copy(acc_sh.at[pl.ds(base, RPT)],
                        agg_hbm.at[cid, pl.ds(base, RPT)])

    return pl.kernel(body, out_type=tuple(out_type), mesh=mesh,
                     scratch_types=scratch,
                     compiler_params=pltpu.CompilerParams(
                         use_tc_tiling_on_sc=False,
                         needs_layout_passes=False))


_sc_agg_counts = _make_sc_agg(True)
_sc_agg = _make_sc_agg(False)


def _bn_body(x_ref, g_ref, b_ref, o_ref):
    x = x_ref[...]
    m = jnp.mean(x, axis=0, keepdims=True)
    xc = x - m
    v = jnp.mean(xc * xc, axis=0, keepdims=True)
    o_ref[...] = xc * lax.rsqrt(v + 1e-5) * g_ref[...] + b_ref[...]


_bn = pl.pallas_call(
    _bn_body, out_shape=jax.ShapeDtypeStruct((N, D), jnp.float32))


def _dense1_body(agg_ref, cnt_ref, h_ref, wlT_ref, bl_ref, wrT_ref, br_ref,
                 g_ref, b_ref, o_ref):
    agg = jnp.concatenate([agg_ref[0, :N], agg_ref[1, :N]], axis=1)
    cnt = cnt_ref[...]
    mean = agg / jnp.maximum(cnt, 1.0)
    t = (jnp.dot(mean, wlT_ref[...], preferred_element_type=jnp.float32)
         + bl_ref[...]
         + jnp.dot(h_ref[...], wrT_ref[...], preferred_element_type=jnp.float32)
         + br_ref[...])
    t = jnp.maximum(t, 0.0)
    m = jnp.mean(t, axis=0, keepdims=True)
    tc = t - m
    v = jnp.mean(tc * tc, axis=0, keepdims=True)
    o_ref[...] = tc * lax.rsqrt(v + 1e-5) * g_ref[...] + b_ref[...]


_dense1 = pl.pallas_call(
    _dense1_body, out_shape=jax.ShapeDtypeStruct((N, D), jnp.float32))


def _dense2_body(agg_ref, cnt_ref, h_ref, wlT_ref, bl_ref, wrT_ref, br_ref,
                 woT_ref, bo_ref, o_ref):
    agg = jnp.concatenate([agg_ref[0, :N], agg_ref[1, :N]], axis=1)
    cnt = cnt_ref[...]
    mean = agg / jnp.maximum(cnt, 1.0)
    t = (jnp.dot(mean, wlT_ref[...], preferred_element_type=jnp.float32)
         + bl_ref[...]
         + jnp.dot(h_ref[...], wrT_ref[...], preferred_element_type=jnp.float32)
         + br_ref[...])
    t = jnp.maximum(t, 0.0)
    z = jnp.dot(t, woT_ref[...], preferred_element_type=jnp.float32) + bo_ref[...]
    o_ref[...] = jax.nn.sigmoid(z)


_dense2 = pl.pallas_call(
    _dense2_body, out_shape=jax.ShapeDtypeStruct((N, 1), jnp.float32))


def kernel(x, edge_index, bn1_g, bn1_b, W_l1, b_l1, W_r1, b_r1,
           bn2_g, bn2_b, W_l2, b_l2, W_r2, b_r2, W_out, b_out):
    srcx = edge_index[0].astype(jnp.int32).reshape(NS, EPT)
    dstx = edge_index[1].astype(jnp.int32).reshape(NS, EPT)
    z_nd = jnp.zeros((NP, HD), jnp.float32)

    h = _bn(x, bn1_g.reshape(1, D), bn1_b.reshape(1, D))
    aggp, cntp = _sc_agg_counts(h.reshape(2 * N, HD), srcx, dstx, z_nd)
    cnt = cntp.sum(axis=0)[:N, None]
    h2 = _dense1(aggp, cnt, h, W_l1.T, b_l1.reshape(1, D),
                 W_r1.T, b_r1.reshape(1, D),
                 bn2_g.reshape(1, D), bn2_b.reshape(1, D))
    (aggp2,) = _sc_agg(h2.reshape(2 * N, HD), srcx, dstx, z_nd)
    out = _dense2(aggp2, cnt, h2, W_l2.T, b_l2.reshape(1, D),
                  W_r2.T, b_r2.reshape(1, D),
                  W_out.T, b_out.reshape(1, 1))
    return out.reshape(N)


# speedup vs baseline: 10.1625x; 1.0073x over previous
import jax
import jax.numpy as jnp
from jax import lax
from jax.experimental import pallas as pl
from jax.experimental.pallas import tpu as pltpu
from jax.experimental.pallas import tpu_sc as plsc

N = 10000
D = 128
E = 320000
HD = D // 2

NC = 2
NS = 16
EPT = E // NS
CHUNK = 40
NCHUNK = EPT // CHUNK
CW = 16
NP = 10240
RPT = NP // NS


def _make_sc_agg(with_counts: bool):
    nbuf = 10
    ngrp = NCHUNK // nbuf
    mesh = plsc.VectorSubcoreMesh(core_axis_name="c", subcore_axis_name="s")
    out_type = [jax.ShapeDtypeStruct((NC, NP, HD), jnp.float32)]
    scratch = [
        pltpu.VMEM((EPT,), jnp.int32),
        pltpu.VMEM((EPT,), jnp.int32),
        pltpu.VMEM((nbuf, CHUNK, HD), jnp.float32),
        pltpu.VMEM((CHUNK, HD), jnp.float32),
        pltpu.VMEM_SHARED((NP, HD), jnp.float32),
        pltpu.SemaphoreType.DMA,
        pltpu.SemaphoreType.DMA,
    ]
    if with_counts:
        out_type.append(jax.ShapeDtypeStruct((NS, NP), jnp.float32))
        scratch.insert(4, pltpu.VMEM((NP,), jnp.float32))

    def body(*refs):
        if with_counts:
            (h_hbm, srcx_hbm, dstx_hbm,
             agg_hbm, cnt_hbm,
             src_v, dst_v, rows_v, zbuf, cnt_t, acc_sh,
             sem_g, sem_s) = refs
        else:
            (h_hbm, srcx_hbm, dstx_hbm,
             agg_hbm,
             src_v, dst_v, rows_v, zbuf, acc_sh, sem_g, sem_s) = refs

        cid = lax.axis_index("c")
        sid = lax.axis_index("s")
        base = sid * RPT

        def zb(i, carry):
            zbuf[i // (HD // 16), pl.ds((i % (HD // 16)) * 16, 16)] = (
                jnp.zeros((16,), jnp.float32))
            return carry

        lax.fori_loop(0, CHUNK * (HD // 16), zb, 0)
        for m in range(RPT // CHUNK):
            pltpu.sync_copy(zbuf, acc_sh.at[pl.ds(base + m * CHUNK, CHUNK)])

        pltpu.sync_copy(srcx_hbm.at[sid], src_v)
        pltpu.sync_copy(dstx_hbm.at[sid], dst_v)

        def xform(i, carry):
            s = src_v[pl.ds(i * 16, 16)]
            src_v[pl.ds(i * 16, 16)] = s + s + cid
            return carry

        lax.fori_loop(0, EPT // 16, xform, 0)

        if with_counts:
            @pl.when(cid == 0)
            def _():
                def czero(i, carry):
                    cnt_t[pl.ds(i * 16, 16)] = jnp.zeros((16,), jnp.float32)
                    return carry

                lax.fori_loop(0, NP // 16, czero, 0)

                def count(i, carry):
                    idx = dst_v[pl.ds(i * 16, 16)]
                    plsc.addupdate_scatter(cnt_t, [idx],
                                           jnp.ones((16,), jnp.float32))
                    return carry

                lax.fori_loop(0, EPT // 16, count, 0)
                pltpu.sync_copy(cnt_t, cnt_hbm.at[sid])

        plsc.subcore_barrier()

        def step(j, carry):
            c0 = j * nbuf
            gathers = []
            for b in range(nbuf):
                gathers.append(pltpu.async_copy(
                    h_hbm.at[src_v.at[pl.ds((c0 + b) * CHUNK, CHUNK)]],
                    rows_v.at[b], sem_g))
            scatters = []
            for b in range(nbuf):
                gathers[b].wait()
                scatters.append(pltpu.async_copy(
                    rows_v.at[b],
                    acc_sh.at[dst_v.at[pl.ds((c0 + b) * CHUNK, CHUNK)]],
                    sem_s, add=True))
            for s in scatters:
                s.wait()
            return carry

        lax.fori_loop(0, ngrp, step, 0)

        plsc.subcore_barrier()
        pltpu.sync_copy(acc_sh.at[pl.ds(base, RPT)],
                        agg_hbm.at[cid, pl.ds(base, RPT)])

    return pl.kernel(body, out_type=tuple(out_type), mesh=mesh,
                     scratch_types=scratch,
                     compiler_params=pltpu.CompilerParams(
                         use_tc_tiling_on_sc=False,
                         needs_layout_passes=False))


_sc_agg_counts = _make_sc_agg(True)
_sc_agg = _make_sc_agg(False)


def _bn_body(x_ref, g_ref, b_ref, o_ref):
    x = x_ref[...]
    m = jnp.mean(x, axis=0, keepdims=True)
    xc = x - m
    v = jnp.mean(xc * xc, axis=0, keepdims=True)
    o_ref[...] = xc * lax.rsqrt(v + 1e-5) * g_ref[...] + b_ref[...]


_bn = pl.pallas_call(
    _bn_body, out_shape=jax.ShapeDtypeStruct((N, D), jnp.float32))


def _lin_body(h_ref, wT_ref, b_ref, o_ref):
    o_ref[...] = jnp.dot(h_ref[...], wT_ref[...],
                         preferred_element_type=jnp.float32) + b_ref[...]


_lin = pl.pallas_call(
    _lin_body, out_shape=jax.ShapeDtypeStruct((N, D), jnp.float32))


def _mean_from_parts(agg_ref, cnt_ref):
    agg = jnp.concatenate([agg_ref[0, :N], agg_ref[1, :N]], axis=1)
    cnt = lax.dot_general(cnt_ref[...], jnp.ones((NS, 1), jnp.float32),
                          (((0,), (0,)), ((), ())),
                          preferred_element_type=jnp.float32)
    return agg / jnp.maximum(cnt[:N], 1.0)


def _mix1_body(agg_ref, cnt_ref, tr_ref, wlT_ref, bl_ref, g_ref, b_ref,
               o_ref):
    mean = _mean_from_parts(agg_ref, cnt_ref)
    t = (jnp.dot(mean, wlT_ref[...], preferred_element_type=jnp.float32)
         + bl_ref[...] + tr_ref[...])
    t = jnp.maximum(t, 0.0)
    m = jnp.mean(t, axis=0, keepdims=True)
    tc = t - m
    v = jnp.mean(tc * tc, axis=0, keepdims=True)
    o_ref[...] = tc * lax.rsqrt(v + 1e-5) * g_ref[...] + b_ref[...]


_mix1 = pl.pallas_call(
    _mix1_body, out_shape=jax.ShapeDtypeStruct((N, D), jnp.float32))


def _mix2_body(agg_ref, cnt_ref, tr_ref, wlT_ref, bl_ref, woT_ref, bo_ref,
               o_ref):
    mean = _mean_from_parts(agg_ref, cnt_ref)
    t = (jnp.dot(mean, wlT_ref[...], preferred_element_type=jnp.float32)
         + bl_ref[...] + tr_ref[...])
    t = jnp.maximum(t, 0.0)
    z = jnp.dot(t, woT_ref[...], preferred_element_type=jnp.float32) + bo_ref[...]
    o_ref[...] = jax.nn.sigmoid(z)


_mix2 = pl.pallas_call(
    _mix2_body, out_shape=jax.ShapeDtypeStruct((N, 1), jnp.float32))


def kernel(x, edge_index, bn1_g, bn1_b, W_l1, b_l1, W_r1, b_r1,
           bn2_g, bn2_b, W_l2, b_l2, W_r2, b_r2, W_out, b_out):
    srcx = edge_index[0].astype(jnp.int32).reshape(NS, EPT)
    dstx = edge_index[1].astype(jnp.int32).reshape(NS, EPT)

    h = _bn(x, bn1_g.reshape(1, D), bn1_b.reshape(1, D))
    tr1 = _lin(h, W_r1.T, b_r1.reshape(1, D))
    aggp, cntp = _sc_agg_counts(h.reshape(2 * N, HD), srcx, dstx)
    h2 = _mix1(aggp, cntp, tr1, W_l1.T, b_l1.reshape(1, D),
               bn2_g.reshape(1, D), bn2_b.reshape(1, D))
    tr2 = _lin(h2, W_r2.T, b_r2.reshape(1, D))
    (aggp2,) = _sc_agg(h2.reshape(2 * N, HD), srcx, dstx)
    out = _mix2(aggp2, cntp, tr2, W_l2.T, b_l2.reshape(1, D),
                W_out.T, b_out.reshape(1, 1))
    return out.reshape(N)


# speedup vs baseline: 10.4921x; 1.0324x over previous
import jax
import jax.numpy as jnp
from jax import lax
from jax.experimental import pallas as pl
from jax.experimental.pallas import tpu as pltpu
from jax.experimental.pallas import tpu_sc as plsc

N = 10000
D = 128
E = 320000
HD = D // 2

NC = 2
NS = 16
EPT = E // NS
CHUNK = 40
NCHUNK = EPT // CHUNK
CW = 16
NP = 10240
RPT = NP // NS


def _make_sc_agg(with_counts: bool):
    nbuf = 10
    ngrp = NCHUNK // nbuf
    mesh = plsc.VectorSubcoreMesh(core_axis_name="c", subcore_axis_name="s")
    out_type = [jax.ShapeDtypeStruct((NC, NP, HD), jnp.float32)]
    scratch = [
        pltpu.VMEM((EPT,), jnp.int32),
        pltpu.VMEM((EPT,), jnp.int32),
        pltpu.VMEM((nbuf, CHUNK, HD), jnp.float32),
        pltpu.VMEM((CHUNK, HD), jnp.float32),
        pltpu.VMEM_SHARED((NP, HD), jnp.float32),
        pltpu.SemaphoreType.DMA,
        pltpu.SemaphoreType.DMA,
    ]
    if with_counts:
        out_type.append(jax.ShapeDtypeStruct((NS, NP), jnp.float32))
        scratch.insert(4, pltpu.VMEM((NP,), jnp.float32))

    def body(*refs):
        if with_counts:
            (h_hbm, edge_hbm,
             agg_hbm, cnt_hbm,
             src_v, dst_v, rows_v, zbuf, cnt_t, acc_sh,
             sem_g, sem_s) = refs
        else:
            (h_hbm, edge_hbm,
             agg_hbm,
             src_v, dst_v, rows_v, zbuf, acc_sh, sem_g, sem_s) = refs

        cid = lax.axis_index("c")
        sid = lax.axis_index("s")
        base = sid * RPT

        def zb(i, carry):
            zbuf[i // (HD // 16), pl.ds((i % (HD // 16)) * 16, 16)] = (
                jnp.zeros((16,), jnp.float32))
            return carry

        lax.fori_loop(0, CHUNK * (HD // 16), zb, 0)
        for m in range(RPT // CHUNK):
            pltpu.sync_copy(zbuf, acc_sh.at[pl.ds(base + m * CHUNK, CHUNK)])

        pltpu.sync_copy(edge_hbm.at[0, pl.ds(sid * EPT, EPT)], src_v)
        pltpu.sync_copy(edge_hbm.at[1, pl.ds(sid * EPT, EPT)], dst_v)

        def xform(i, carry):
            s = src_v[pl.ds(i * 16, 16)]
            src_v[pl.ds(i * 16, 16)] = s + s + cid
            return carry

        lax.fori_loop(0, EPT // 16, xform, 0)

        if with_counts:
            @pl.when(cid == 0)
            def _():
                def czero(i, carry):
                    cnt_t[pl.ds(i * 16, 16)] = jnp.zeros((16,), jnp.float32)
                    return carry

                lax.fori_loop(0, NP // 16, czero, 0)

                def count(i, carry):
                    idx = dst_v[pl.ds(i * 16, 16)]
                    plsc.addupdate_scatter(cnt_t, [idx],
                                           jnp.ones((16,), jnp.float32))
                    return carry

                lax.fori_loop(0, EPT // 16, count, 0)
                pltpu.sync_copy(cnt_t, cnt_hbm.at[sid])

        plsc.subcore_barrier()

        def step(j, carry):
            c0 = j * nbuf
            gathers = []
            for b in range(nbuf):
                gathers.append(pltpu.async_copy(
                    h_hbm.at[src_v.at[pl.ds((c0 + b) * CHUNK, CHUNK)]],
                    rows_v.at[b], sem_g))
            scatters = []
            for b in range(nbuf):
                gathers[b].wait()
                scatters.append(pltpu.async_copy(
                    rows_v.at[b],
                    acc_sh.at[dst_v.at[pl.ds((c0 + b) * CHUNK, CHUNK)]],
                    sem_s, add=True))
            for s in scatters:
                s.wait()
            return carry

        lax.fori_loop(0, ngrp, step, 0)

        plsc.subcore_barrier()
        pltpu.sync_copy(acc_sh.at[pl.ds(base, RPT)],
                        agg_hbm.at[cid, pl.ds(base, RPT)])

    return pl.kernel(body, out_type=tuple(out_type), mesh=mesh,
                     scratch_types=scratch,
                     compiler_params=pltpu.CompilerParams(
                         use_tc_tiling_on_sc=False,
                         needs_layout_passes=False))


_sc_agg_counts = _make_sc_agg(True)
_sc_agg = _make_sc_agg(False)


def _bn_body(x_ref, g_ref, b_ref, o_ref):
    x = x_ref[...]
    m = jnp.mean(x, axis=0, keepdims=True)
    xc = x - m
    v = jnp.mean(xc * xc, axis=0, keepdims=True)
    o_ref[...] = xc * lax.rsqrt(v + 1e-5) * g_ref[...] + b_ref[...]


_bn = pl.pallas_call(
    _bn_body, out_shape=jax.ShapeDtypeStruct((N, D), jnp.float32))


def _mean_from_parts(agg_ref, cnt_ref):
    agg = jnp.concatenate([agg_ref[0, :N], agg_ref[1, :N]], axis=1)
    cnt = lax.dot_general(cnt_ref[...], jnp.ones((NS, 1), jnp.float32),
                          (((0,), (0,)), ((), ())),
                          preferred_element_type=jnp.float32)
    return agg / jnp.maximum(cnt[:N], 1.0)


def _mix1_body(agg_ref, cnt_ref, h_ref, wlT_ref, bl_ref, wrT_ref, br_ref,
               g_ref, b_ref, o_ref):
    mean = _mean_from_parts(agg_ref, cnt_ref)
    t = (jnp.dot(mean, wlT_ref[...], preferred_element_type=jnp.float32)
         + bl_ref[...]
         + jnp.dot(h_ref[...], wrT_ref[...], preferred_element_type=jnp.float32)
         + br_ref[...])
    t = jnp.maximum(t, 0.0)
    m = jnp.mean(t, axis=0, keepdims=True)
    tc = t - m
    v = jnp.mean(tc * tc, axis=0, keepdims=True)
    o_ref[...] = tc * lax.rsqrt(v + 1e-5) * g_ref[...] + b_ref[...]


_mix1 = pl.pallas_call(
    _mix1_body, out_shape=jax.ShapeDtypeStruct((N, D), jnp.float32))


def _mix2_body(agg_ref, cnt_ref, h_ref, wlT_ref, bl_ref, wrT_ref, br_ref,
               woT_ref, bo_ref, o_ref):
    mean = _mean_from_parts(agg_ref, cnt_ref)
    t = (jnp.dot(mean, wlT_ref[...], preferred_element_type=jnp.float32)
         + bl_ref[...]
         + jnp.dot(h_ref[...], wrT_ref[...], preferred_element_type=jnp.float32)
         + br_ref[...])
    t = jnp.maximum(t, 0.0)
    z = jnp.dot(t, woT_ref[...], preferred_element_type=jnp.float32) + bo_ref[...]
    o_ref[...] = jax.nn.sigmoid(z)


_mix2 = pl.pallas_call(
    _mix2_body, out_shape=jax.ShapeDtypeStruct((N, 1), jnp.float32))


def kernel(x, edge_index, bn1_g, bn1_b, W_l1, b_l1, W_r1, b_r1,
           bn2_g, bn2_b, W_l2, b_l2, W_r2, b_r2, W_out, b_out):
    edges = edge_index.astype(jnp.int32)

    h = _bn(x, bn1_g.reshape(1, D), bn1_b.reshape(1, D))
    aggp, cntp = _sc_agg_counts(h.reshape(2 * N, HD), edges)
    h2 = _mix1(aggp, cntp, h, W_l1.T, b_l1.reshape(1, D),
               W_r1.T, b_r1.reshape(1, D),
               bn2_g.reshape(1, D), bn2_b.reshape(1, D))
    (aggp2,) = _sc_agg(h2.reshape(2 * N, HD), edges)
    out = _mix2(aggp2, cntp, h2, W_l2.T, b_l2.reshape(1, D),
                W_r2.T, b_r2.reshape(1, D),
                W_out.T, b_out.reshape(1, 1))
    return out.reshape(N)


# speedup vs baseline: 10.4923x; 1.0000x over previous
import jax
import jax.numpy as jnp
from jax import lax
from jax.experimental import pallas as pl
from jax.experimental.pallas import tpu as pltpu
from jax.experimental.pallas import tpu_sc as plsc

N = 10000
D = 128
E = 320000
HD = D // 2

NC = 2
NS = 16
EPT = E // NS
CHUNK = 40
NCHUNK = EPT // CHUNK
NP = 10240
RPT = NP // NS


def _make_sc_agg(with_counts: bool):
    nbuf = 10
    ngrp = NCHUNK // nbuf
    mesh = plsc.VectorSubcoreMesh(core_axis_name="c", subcore_axis_name="s")
    out_type = [jax.ShapeDtypeStruct((NC, NP, HD), jnp.float32)]
    scratch = [
        pltpu.VMEM((EPT,), jnp.int32),
        pltpu.VMEM((EPT,), jnp.int32),
        pltpu.VMEM((nbuf, CHUNK, HD), jnp.float32),
        pltpu.VMEM((CHUNK, HD), jnp.float32),
        pltpu.VMEM_SHARED((NP, HD), jnp.float32),
        pltpu.SemaphoreType.DMA,
        pltpu.SemaphoreType.DMA,
    ]
    if with_counts:
        out_type.append(jax.ShapeDtypeStruct((NS, NP), jnp.float32))
        scratch.insert(4, pltpu.VMEM((NP,), jnp.float32))

    def body(*refs):
        if with_counts:
            (h_hbm, edge_hbm,
             agg_hbm, cnt_hbm,
             src_v, dst_v, rows_v, zbuf, cnt_t, acc_sh,
             sem_g, sem_s) = refs
        else:
            (h_hbm, edge_hbm,
             agg_hbm,
             src_v, dst_v, rows_v, zbuf, acc_sh, sem_g, sem_s) = refs

        cid = lax.axis_index("c")
        sid = lax.axis_index("s")
        base = sid * RPT

        def zb(i, carry):
            zbuf[i // (HD // 16), pl.ds((i % (HD // 16)) * 16, 16)] = (
                jnp.zeros((16,), jnp.float32))
            return carry

        lax.fori_loop(0, CHUNK * (HD // 16), zb, 0)
        for m in range(RPT // CHUNK):
            pltpu.sync_copy(zbuf, acc_sh.at[pl.ds(base + m * CHUNK, CHUNK)])

        pltpu.sync_copy(edge_hbm.at[0, pl.ds(sid * EPT, EPT)], src_v)
        pltpu.sync_copy(edge_hbm.at[1, pl.ds(sid * EPT, EPT)], dst_v)

        def xform(i, carry):
            s = src_v[pl.ds(i * 16, 16)]
            src_v[pl.ds(i * 16, 16)] = s + s + cid
            return carry

        lax.fori_loop(0, EPT // 16, xform, 0)

        if with_counts:
            @pl.when(cid == 0)
            def _():
                def czero(i, carry):
                    cnt_t[pl.ds(i * 16, 16)] = jnp.zeros((16,), jnp.float32)
                    return carry

                lax.fori_loop(0, NP // 16, czero, 0)

                def count(i, carry):
                    idx = dst_v[pl.ds(i * 16, 16)]
                    plsc.addupdate_scatter(cnt_t, [idx],
                                           jnp.ones((16,), jnp.float32))
                    return carry

                lax.fori_loop(0, EPT // 16, count, 0)
                pltpu.sync_copy(cnt_t, cnt_hbm.at[sid])

        plsc.subcore_barrier()

        def step(j, carry):
            c0 = j * nbuf
            gathers = []
            for b in range(nbuf):
                gathers.append(pltpu.async_copy(
                    h_hbm.at[src_v.at[pl.ds((c0 + b) * CHUNK, CHUNK)]],
                    rows_v.at[b], sem_g))
            scatters = []
            for b in range(nbuf):
                gathers[b].wait()
                scatters.append(pltpu.async_copy(
                    rows_v.at[b],
                    acc_sh.at[dst_v.at[pl.ds((c0 + b) * CHUNK, CHUNK)]],
                    sem_s, add=True))
            for s in scatters:
                s.wait()
            return carry

        lax.fori_loop(0, ngrp, step, 0)

        plsc.subcore_barrier()
        pltpu.sync_copy(acc_sh.at[pl.ds(base, RPT)],
                        agg_hbm.at[cid, pl.ds(base, RPT)])

    return pl.kernel(body, out_type=tuple(out_type), mesh=mesh,
                     scratch_types=scratch,
                     compiler_params=pltpu.CompilerParams(
                         use_tc_tiling_on_sc=False,
                         needs_layout_passes=False))


_sc_agg_counts = _make_sc_agg(True)
_sc_agg = _make_sc_agg(False)


def _bn_body(x_ref, g_ref, b_ref, o_ref):
    x = x_ref[...]
    m = jnp.mean(x, axis=0, keepdims=True)
    xc = x - m
    v = jnp.mean(xc * xc, axis=0, keepdims=True)
    o_ref[...] = xc * lax.rsqrt(v + 1e-5) * g_ref[...] + b_ref[...]


_bn = pl.pallas_call(
    _bn_body, out_shape=jax.ShapeDtypeStruct((N, D), jnp.float32))


def _mean_from_parts(agg_ref, cnt_ref):
    agg = jnp.concatenate([agg_ref[0, :N], agg_ref[1, :N]], axis=1)
    cnt = lax.dot_general(cnt_ref[...], jnp.ones((NS, 1), jnp.float32),
                          (((0,), (0,)), ((), ())),
                          preferred_element_type=jnp.float32)
    return agg / jnp.maximum(cnt[:N], 1.0)


def _mix1_body(agg_ref, cnt_ref, h_ref, wlT_ref, bl_ref, wrT_ref, br_ref,
               g_ref, b_ref, o_ref):
    mean = _mean_from_parts(agg_ref, cnt_ref)
    t = (jnp.dot(mean, wlT_ref[...], preferred_element_type=jnp.float32)
         + bl_ref[...]
         + jnp.dot(h_ref[...], wrT_ref[...], preferred_element_type=jnp.float32)
         + br_ref[...])
    t = jnp.maximum(t, 0.0)
    m = jnp.mean(t, axis=0, keepdims=True)
    tc = t - m
    v = jnp.mean(tc * tc, axis=0, keepdims=True)
    o_ref[...] = tc * lax.rsqrt(v + 1e-5) * g_ref[...] + b_ref[...]


_mix1 = pl.pallas_call(
    _mix1_body, out_shape=jax.ShapeDtypeStruct((N, D), jnp.float32))


def _mix2_body(agg_ref, cnt_ref, h_ref, wlT_ref, bl_ref, wrT_ref, br_ref,
               woT_ref, bo_ref, o_ref):
    mean = _mean_from_parts(agg_ref, cnt_ref)
    t = (jnp.dot(mean, wlT_ref[...], preferred_element_type=jnp.float32)
         + bl_ref[...]
         + jnp.dot(h_ref[...], wrT_ref[...], preferred_element_type=jnp.float32)
         + br_ref[...])
    t = jnp.maximum(t, 0.0)
    z = jnp.dot(t, woT_ref[...], preferred_element_type=jnp.float32) + bo_ref[...]
    o_ref[...] = jax.nn.sigmoid(z)


_mix2 = pl.pallas_call(
    _mix2_body, out_shape=jax.ShapeDtypeStruct((N, 1), jnp.float32))


def kernel(x, edge_index, bn1_g, bn1_b, W_l1, b_l1, W_r1, b_r1,
           bn2_g, bn2_b, W_l2, b_l2, W_r2, b_r2, W_out, b_out):
    edges = edge_index.astype(jnp.int32)

    h = _bn(x, bn1_g.reshape(1, D), bn1_b.reshape(1, D))
    aggp, cntp = _sc_agg_counts(h.reshape(2 * N, HD), edges)
    h2 = _mix1(aggp, cntp, h, W_l1.T, b_l1.reshape(1, D),
               W_r1.T, b_r1.reshape(1, D),
               bn2_g.reshape(1, D), bn2_b.reshape(1, D))
    (aggp2,) = _sc_agg(h2.reshape(2 * N, HD), edges)
    out = _mix2(aggp2, cntp, h2, W_l2.T, b_l2.reshape(1, D),
                W_r2.T, b_r2.reshape(1, D),
                W_out.T, b_out.reshape(1, 1))
    return out.reshape(N)
